# Initial kernel scaffold; baseline (speedup 1.0000x reference)
import jax
import jax.numpy as jnp
from jax.experimental import pallas as pl


def kernel(x, edge_index, W1, b1, W2, b2, Wfc1, bfc1, Wfc2, bfc2):
    raise NotImplementedError("write your pallas kernel here")


# speedup vs baseline: 4.2878x; 4.2878x over previous
import functools

import jax
import jax.numpy as jnp
from jax import lax
from jax.experimental import pallas as pl
from jax.experimental.pallas import tpu as pltpu
from jax.experimental.pallas import tpu_sc as plsc

NN = 10000
NE = 320000
ET = NE + NN
NPAD = 10112
DUMMY = NN
GROUP = 128
STAGE = 8
G1 = 168
G2 = 88
EP1 = 16 * G1 * GROUP
EP2 = 32 * G2 * GROUP
ROWS_PER_TILE = NPAD // 16
_BLK = 1000


def _sc_mesh():
    return plsc.VectorSubcoreMesh(core_axis_name="c", subcore_axis_name="s",
                                  num_cores=2, num_subcores=16)


def _zero_acc_rows(acc, zbuf, r0):
    for t in range(ROWS_PER_TILE // GROUP):
        pltpu.sync_copy(zbuf, acc.at[pl.ds(r0 + t * GROUP, GROUP)])
    rem = ROWS_PER_TILE % GROUP
    if rem:
        base = r0 + (ROWS_PER_TILE // GROUP) * GROUP
        pltpu.sync_copy(zbuf.at[pl.ds(0, rem)], acc.at[pl.ds(base, rem)])


def _make_agg(n_table_rows, n_groups):
    @functools.partial(
        pl.kernel,
        out_type=jax.ShapeDtypeStruct((2, NPAD, 128), jnp.float32),
        mesh=_sc_mesh(),
        scratch_types=[
            pltpu.VMEM((STAGE, GROUP), jnp.int32),
            pltpu.VMEM((STAGE, GROUP), jnp.int32),
            pltpu.VMEM((GROUP, 128), jnp.float32),
            pltpu.VMEM_SHARED((NPAD, 128), jnp.float32),
            pltpu.SemaphoreType.DMA,
            pltpu.SemaphoreType.DMA,
        ],
    )
    def agg(table_hbm, zeros_hbm, src_hbm, dst_hbm, out_hbm,
            src_v, dst_v, val_v, acc, gsem, ssem):
        c = lax.axis_index("c")
        s = lax.axis_index("s")
        r0 = s * ROWS_PER_TILE
        pltpu.sync_copy(zeros_hbm, val_v)
        _zero_acc_rows(acc, val_v, r0)
        plsc.subcore_barrier()

        def outer(o, carry):
            pltpu.sync_copy(src_hbm.at[c, s, o], src_v)
            pltpu.sync_copy(dst_hbm.at[c, s, o], dst_v)

            def body(g, carry2):
                pltpu.async_copy(table_hbm.at[src_v.at[g]], val_v, gsem).wait()
                pltpu.async_copy(val_v, acc.at[dst_v.at[g]], ssem,
                                 add=True).wait()
                return carry2

            lax.fori_loop(0, STAGE, body, 0)
            return carry

        lax.fori_loop(0, n_groups // STAGE, outer, 0)
        plsc.subcore_barrier()
        pltpu.sync_copy(acc.at[pl.ds(r0, ROWS_PER_TILE)],
                        out_hbm.at[c].at[pl.ds(r0, ROWS_PER_TILE)])

    return agg


def _make_deg(n_groups):
    @functools.partial(
        pl.kernel,
        out_type=jax.ShapeDtypeStruct((2, NPAD, 128), jnp.float32),
        mesh=_sc_mesh(),
        scratch_types=[
            pltpu.VMEM((STAGE, GROUP), jnp.int32),
            pltpu.VMEM((GROUP, 128), jnp.float32),
            pltpu.VMEM((GROUP, 128), jnp.float32),
            pltpu.VMEM_SHARED((NPAD, 128), jnp.float32),
            pltpu.SemaphoreType.DMA,
        ],
    )
    def deg(ones_hbm, zeros_hbm, dst_hbm, out_hbm,
            dst_v, ones_v, zero_v, acc, ssem):
        c = lax.axis_index("c")
        s = lax.axis_index("s")
        r0 = s * ROWS_PER_TILE
        pltpu.sync_copy(zeros_hbm, zero_v)
        _zero_acc_rows(acc, zero_v, r0)
        pltpu.sync_copy(ones_hbm, ones_v)
        plsc.subcore_barrier()

        def outer(o, carry):
            pltpu.sync_copy(dst_hbm.at[c, s, o], dst_v)

            def body(g, carry2):
                pltpu.async_copy(ones_v, acc.at[dst_v.at[g]], ssem,
                                 add=True).wait()
                return carry2

            lax.fori_loop(0, STAGE, body, 0)
            return carry

        lax.fori_loop(0, n_groups // STAGE, outer, 0)
        plsc.subcore_barrier()
        pltpu.sync_copy(acc.at[pl.ds(r0, ROWS_PER_TILE)],
                        out_hbm.at[c].at[pl.ds(r0, ROWS_PER_TILE)])

    return deg


def _dis_of(deg_ref):
    return lax.rsqrt(jnp.maximum(deg_ref[:, 0:1], 1.0))


def _tc1_body(deg_ref, x_ref, w_ref, o_ref):
    dis = _dis_of(deg_ref)
    o_ref[...] = jnp.dot(x_ref[...] * dis, w_ref[...],
                         preferred_element_type=jnp.float32)


def _tc2_body(deg_ref, a_ref, b1_ref, w_ref, o_ref):
    dis = _dis_of(deg_ref)
    h = jnp.maximum(a_ref[...] * dis + b1_ref[...], 0.0)
    o_ref[...] = jnp.dot(h, w_ref[...],
                         preferred_element_type=jnp.float32) * dis


def _tc3_body(deg_ref, a_ref, b2_ref, w1_ref, c1_ref, w2_ref, c2_ref, o_ref):
    dis = _dis_of(deg_ref)
    h = jnp.maximum(a_ref[...] * dis + b2_ref[...], 0.0)
    p = jnp.dot(h, w1_ref[...], preferred_element_type=jnp.float32) + c1_ref[...]
    p = jnp.where(p > 0.0, p, jnp.exp(p) - 1.0)
    o_ref[...] = jnp.dot(p, w2_ref[...],
                         preferred_element_type=jnp.float32) + c2_ref[...]


def _row_spec(cols):
    return pl.BlockSpec((_BLK, cols), lambda i: (i, 0))


def _full_spec(r, c):
    return pl.BlockSpec((r, c), lambda i: (0, 0))


def _pad_edges(src, dst, total):
    npad_e = total - ET
    srcp = jnp.concatenate([src, jnp.zeros((npad_e,), jnp.int32)])
    dstp = jnp.concatenate([dst, jnp.full((npad_e,), DUMMY, jnp.int32)])
    return srcp, dstp


def kernel(x, edge_index, W1, b1, W2, b2, Wfc1, bfc1, Wfc2, bfc2):
    i32 = jnp.int32
    f32 = jnp.float32
    loop = jnp.arange(NN, dtype=i32)
    src = jnp.concatenate([edge_index[0], loop])
    dst = jnp.concatenate([edge_index[1], loop])

    srcp1, dstp1 = _pad_edges(src, dst, EP1)
    src1 = jnp.stack([srcp1, srcp1 + NN]).reshape(2, 16, G1 // STAGE, STAGE, GROUP)
    dst1 = jnp.broadcast_to(dstp1, (2, EP1)).reshape(2, 16, G1 // STAGE, STAGE, GROUP)
    srcp2, dstp2 = _pad_edges(src, dst, EP2)
    src2 = srcp2.reshape(2, 16, G2 // STAGE, STAGE, GROUP)
    dst2 = dstp2.reshape(2, 16, G2 // STAGE, STAGE, GROUP)

    zeros128 = jnp.zeros((GROUP, 128), f32)
    ones128 = jnp.ones((GROUP, 128), f32)

    degp = _make_deg(G2)(ones128, zeros128, dst2)
    degs = (degp[0, :NN, :16] + degp[1, :NN, :16])

    xs1 = pl.pallas_call(
        _tc1_body,
        grid=(NN // _BLK,),
        in_specs=[_row_spec(16), _row_spec(128), _full_spec(128, 256)],
        out_specs=_row_spec(256),
        out_shape=jax.ShapeDtypeStruct((NN, 256), f32),
    )(degs, x, W1)

    table1 = jnp.concatenate([xs1[:, :128], xs1[:, 128:]], axis=0)
    agg1 = _make_agg(2 * NN, G1)(table1, zeros128, src1, dst1)
    agg1c = jnp.concatenate([agg1[0, :NN], agg1[1, :NN]], axis=1)

    xs2 = pl.pallas_call(
        _tc2_body,
        grid=(NN // _BLK,),
        in_specs=[_row_spec(16), _row_spec(256), _full_spec(1, 256),
                  _full_spec(256, 128)],
        out_specs=_row_spec(128),
        out_shape=jax.ShapeDtypeStruct((NN, 128), f32),
    )(degs, agg1c, b1.reshape(1, 256), W2)

    agg2 = _make_agg(NN, G2)(xs2, zeros128, src2, dst2)
    agg2s = agg2[0, :NN] + agg2[1, :NN]

    out = pl.pallas_call(
        _tc3_body,
        grid=(NN // _BLK,),
        in_specs=[_row_spec(16), _row_spec(128), _full_spec(1, 128),
                  _full_spec(128, 128), _full_spec(1, 128),
                  _full_spec(128, 128), _full_spec(1, 128)],
        out_specs=_row_spec(128),
        out_shape=jax.ShapeDtypeStruct((NN, 128), f32),
    )(degs, agg2s, b2.reshape(1, 128), Wfc1, bfc1.reshape(1, 128),
      Wfc2, bfc2.reshape(1, 128))
    return out


# speedup vs baseline: 4.3974x; 1.0256x over previous
import functools

import jax
import jax.numpy as jnp
from jax import lax
from jax.experimental import pallas as pl
from jax.experimental.pallas import tpu as pltpu
from jax.experimental.pallas import tpu_sc as plsc

NN = 10000
NE = 320000
ET = NE + NN
NPAD = 10112
DUMMY = NN
GROUP = 128
STAGE = 8
G1 = 168
G2 = 88
EP1 = 16 * G1 * GROUP
EP2 = 32 * G2 * GROUP
ROWS_PER_TILE = NPAD // 16
_BLK = 1000


def _sc_mesh():
    return plsc.VectorSubcoreMesh(core_axis_name="c", subcore_axis_name="s",
                                  num_cores=2, num_subcores=16)


def _zero_acc_rows(acc, zbuf, r0):
    for t in range(ROWS_PER_TILE // GROUP):
        pltpu.sync_copy(zbuf, acc.at[pl.ds(r0 + t * GROUP, GROUP)])
    rem = ROWS_PER_TILE % GROUP
    if rem:
        base = r0 + (ROWS_PER_TILE // GROUP) * GROUP
        pltpu.sync_copy(zbuf.at[pl.ds(0, rem)], acc.at[pl.ds(base, rem)])


def _make_agg(n_table_rows, n_groups):
    @functools.partial(
        pl.kernel,
        out_type=jax.ShapeDtypeStruct((2, NPAD, 128), jnp.float32),
        mesh=_sc_mesh(),
        scratch_types=[
            pltpu.VMEM((STAGE, GROUP), jnp.int32),
            pltpu.VMEM((STAGE, GROUP), jnp.int32),
            pltpu.VMEM((GROUP, 128), jnp.float32),
            pltpu.VMEM((GROUP, 128), jnp.float32),
            pltpu.VMEM_SHARED((NPAD, 128), jnp.float32),
            pltpu.SemaphoreType.DMA,
            pltpu.SemaphoreType.DMA,
            pltpu.SemaphoreType.DMA,
            pltpu.SemaphoreType.DMA,
        ],
    )
    def agg(table_hbm, zeros_hbm, src_hbm, dst_hbm, out_hbm,
            src_v, dst_v, val0, val1, acc, gsem0, gsem1, ssem0, ssem1):
        c = lax.axis_index("c")
        s = lax.axis_index("s")
        r0 = s * ROWS_PER_TILE
        pltpu.sync_copy(zeros_hbm, val0)
        _zero_acc_rows(acc, val0, r0)
        plsc.subcore_barrier()
        vals = (val0, val1)
        gsems = (gsem0, gsem1)
        ssems = (ssem0, ssem1)

        def outer(o, carry):
            pltpu.sync_copy(src_hbm.at[c, s, o], src_v)
            pltpu.sync_copy(dst_hbm.at[c, s, o], dst_v)
            sd = [None] * STAGE
            for g in range(STAGE):
                b = g & 1
                if g >= 2:
                    sd[g - 2].wait()
                gd = pltpu.async_copy(table_hbm.at[src_v.at[g]], vals[b],
                                      gsems[b])
                gd.wait()
                sd[g] = pltpu.async_copy(vals[b], acc.at[dst_v.at[g]],
                                         ssems[b], add=True)
            sd[STAGE - 2].wait()
            sd[STAGE - 1].wait()
            return carry

        lax.fori_loop(0, n_groups // STAGE, outer, 0)
        plsc.subcore_barrier()
        pltpu.sync_copy(acc.at[pl.ds(r0, ROWS_PER_TILE)],
                        out_hbm.at[c].at[pl.ds(r0, ROWS_PER_TILE)])

    return agg


def _make_deg(n_groups):
    @functools.partial(
        pl.kernel,
        out_type=jax.ShapeDtypeStruct((2, NPAD, 128), jnp.float32),
        mesh=_sc_mesh(),
        scratch_types=[
            pltpu.VMEM((STAGE, GROUP), jnp.int32),
            pltpu.VMEM((GROUP, 128), jnp.float32),
            pltpu.VMEM((GROUP, 128), jnp.float32),
            pltpu.VMEM_SHARED((NPAD, 128), jnp.float32),
            pltpu.SemaphoreType.DMA,
        ],
    )
    def deg(ones_hbm, zeros_hbm, dst_hbm, out_hbm,
            dst_v, ones_v, zero_v, acc, ssem):
        c = lax.axis_index("c")
        s = lax.axis_index("s")
        r0 = s * ROWS_PER_TILE
        pltpu.sync_copy(zeros_hbm, zero_v)
        _zero_acc_rows(acc, zero_v, r0)
        pltpu.sync_copy(ones_hbm, ones_v)
        plsc.subcore_barrier()

        def outer(o, carry):
            pltpu.sync_copy(dst_hbm.at[c, s, o], dst_v)
            sd = [pltpu.async_copy(ones_v, acc.at[dst_v.at[g]], ssem,
                                   add=True)
                  for g in range(STAGE)]
            for d in sd:
                d.wait()
            return carry

        lax.fori_loop(0, n_groups // STAGE, outer, 0)
        plsc.subcore_barrier()
        pltpu.sync_copy(acc.at[pl.ds(r0, ROWS_PER_TILE)],
                        out_hbm.at[c].at[pl.ds(r0, ROWS_PER_TILE)])

    return deg


def _dis_of(deg_ref):
    return lax.rsqrt(jnp.maximum(deg_ref[:, 0:1], 1.0))


def _tc1_body(deg_ref, x_ref, w_ref, o_ref):
    dis = _dis_of(deg_ref)
    o_ref[...] = jnp.dot(x_ref[...] * dis, w_ref[...],
                         preferred_element_type=jnp.float32)


def _tc2_body(deg_ref, a_ref, b1_ref, w_ref, o_ref):
    dis = _dis_of(deg_ref)
    h = jnp.maximum(a_ref[...] * dis + b1_ref[...], 0.0)
    o_ref[...] = jnp.dot(h, w_ref[...],
                         preferred_element_type=jnp.float32) * dis


def _tc3_body(deg_ref, a_ref, b2_ref, w1_ref, c1_ref, w2_ref, c2_ref, o_ref):
    dis = _dis_of(deg_ref)
    h = jnp.maximum(a_ref[...] * dis + b2_ref[...], 0.0)
    p = jnp.dot(h, w1_ref[...], preferred_element_type=jnp.float32) + c1_ref[...]
    p = jnp.where(p > 0.0, p, jnp.exp(p) - 1.0)
    o_ref[...] = jnp.dot(p, w2_ref[...],
                         preferred_element_type=jnp.float32) + c2_ref[...]


def _row_spec(cols):
    return pl.BlockSpec((_BLK, cols), lambda i: (i, 0))


def _full_spec(r, c):
    return pl.BlockSpec((r, c), lambda i: (0, 0))


def _pad_edges(src, dst, total):
    npad_e = total - ET
    srcp = jnp.concatenate([src, jnp.zeros((npad_e,), jnp.int32)])
    dstp = jnp.concatenate([dst, jnp.full((npad_e,), DUMMY, jnp.int32)])
    return srcp, dstp


def kernel(x, edge_index, W1, b1, W2, b2, Wfc1, bfc1, Wfc2, bfc2):
    i32 = jnp.int32
    f32 = jnp.float32
    loop = jnp.arange(NN, dtype=i32)
    src = jnp.concatenate([edge_index[0], loop])
    dst = jnp.concatenate([edge_index[1], loop])

    srcp1, dstp1 = _pad_edges(src, dst, EP1)
    src1 = jnp.stack([srcp1, srcp1 + NN]).reshape(2, 16, G1 // STAGE, STAGE, GROUP)
    dst1 = jnp.broadcast_to(dstp1, (2, EP1)).reshape(2, 16, G1 // STAGE, STAGE, GROUP)
    srcp2, dstp2 = _pad_edges(src, dst, EP2)
    src2 = srcp2.reshape(2, 16, G2 // STAGE, STAGE, GROUP)
    dst2 = dstp2.reshape(2, 16, G2 // STAGE, STAGE, GROUP)

    zeros128 = jnp.zeros((GROUP, 128), f32)
    ones128 = jnp.ones((GROUP, 128), f32)

    degp = _make_deg(G2)(ones128, zeros128, dst2)
    degs = (degp[0, :NN, :16] + degp[1, :NN, :16])

    xs1 = pl.pallas_call(
        _tc1_body,
        grid=(NN // _BLK,),
        in_specs=[_row_spec(16), _row_spec(128), _full_spec(128, 256)],
        out_specs=_row_spec(256),
        out_shape=jax.ShapeDtypeStruct((NN, 256), f32),
    )(degs, x, W1)

    table1 = jnp.concatenate([xs1[:, :128], xs1[:, 128:]], axis=0)
    agg1 = _make_agg(2 * NN, G1)(table1, zeros128, src1, dst1)
    agg1c = jnp.concatenate([agg1[0, :NN], agg1[1, :NN]], axis=1)

    xs2 = pl.pallas_call(
        _tc2_body,
        grid=(NN // _BLK,),
        in_specs=[_row_spec(16), _row_spec(256), _full_spec(1, 256),
                  _full_spec(256, 128)],
        out_specs=_row_spec(128),
        out_shape=jax.ShapeDtypeStruct((NN, 128), f32),
    )(degs, agg1c, b1.reshape(1, 256), W2)

    agg2 = _make_agg(NN, G2)(xs2, zeros128, src2, dst2)
    agg2s = agg2[0, :NN] + agg2[1, :NN]

    out = pl.pallas_call(
        _tc3_body,
        grid=(NN // _BLK,),
        in_specs=[_row_spec(16), _row_spec(128), _full_spec(1, 128),
                  _full_spec(128, 128), _full_spec(1, 128),
                  _full_spec(128, 128), _full_spec(1, 128)],
        out_specs=_row_spec(128),
        out_shape=jax.ShapeDtypeStruct((NN, 128), f32),
    )(degs, agg2s, b2.reshape(1, 128), Wfc1, bfc1.reshape(1, 128),
      Wfc2, bfc2.reshape(1, 128))
    return out


# speedup vs baseline: 16.9480x; 3.8541x over previous
import functools

import jax
import jax.numpy as jnp
from jax import lax
from jax.experimental import pallas as pl
from jax.experimental.pallas import tpu as pltpu
from jax.experimental.pallas import tpu_sc as plsc

NN = 10000
NE = 320000
ET = NE + NN
NPAD = 10112
DUMMY = NN
GROUP = 128
STAGE = 8
G1 = 168
G2 = 88
EP1 = 16 * G1 * GROUP
EP2 = 32 * G2 * GROUP
ROWS_PER_TILE = NPAD // 16
ZPAD = 128
_BLK = 1000


def _sc_mesh():
    return plsc.VectorSubcoreMesh(core_axis_name="c", subcore_axis_name="s",
                                  num_cores=2, num_subcores=16)


def _zero_acc_rows(acc, zbuf, r0):
    for t in range(ROWS_PER_TILE // GROUP):
        pltpu.sync_copy(zbuf, acc.at[pl.ds(r0 + t * GROUP, GROUP)])
    rem = ROWS_PER_TILE % GROUP
    if rem:
        base = r0 + (ROWS_PER_TILE // GROUP) * GROUP
        pltpu.sync_copy(zbuf.at[pl.ds(0, rem)], acc.at[pl.ds(base, rem)])


def _make_agg(n_table_rows, n_groups):
    @functools.partial(
        pl.kernel,
        out_type=jax.ShapeDtypeStruct((2, NPAD, 128), jnp.float32),
        mesh=_sc_mesh(),
        scratch_types=[
            pltpu.VMEM((STAGE, GROUP), jnp.int32),
            pltpu.VMEM((STAGE, GROUP), jnp.int32),
            pltpu.VMEM((GROUP, 128), jnp.float32),
            pltpu.VMEM((GROUP, 128), jnp.float32),
            pltpu.VMEM_SHARED((NPAD, 128), jnp.float32),
            pltpu.SemaphoreType.DMA,
            pltpu.SemaphoreType.DMA,
            pltpu.SemaphoreType.DMA,
            pltpu.SemaphoreType.DMA,
        ],
    )
    def agg(table_hbm, zeros_hbm, src_hbm, dst_hbm, out_hbm,
            src_v, dst_v, val0, val1, acc, gsem0, gsem1, ssem0, ssem1):
        c = lax.axis_index("c")
        s = lax.axis_index("s")
        r0 = s * ROWS_PER_TILE
        pltpu.sync_copy(zeros_hbm, val0)
        _zero_acc_rows(acc, val0, r0)
        plsc.subcore_barrier()
        vals = (val0, val1)
        gsems = (gsem0, gsem1)
        ssems = (ssem0, ssem1)

        def outer(o, carry):
            pltpu.sync_copy(src_hbm.at[c, s, o], src_v)
            pltpu.sync_copy(dst_hbm.at[c, s, o], dst_v)
            sd = [None] * STAGE
            for g in range(STAGE):
                b = g & 1
                if g >= 2:
                    sd[g - 2].wait()
                gd = pltpu.async_copy(table_hbm.at[src_v.at[g]], vals[b],
                                      gsems[b])
                gd.wait()
                sd[g] = pltpu.async_copy(vals[b], acc.at[dst_v.at[g]],
                                         ssems[b], add=True)
            sd[STAGE - 2].wait()
            sd[STAGE - 1].wait()
            return carry

        lax.fori_loop(0, n_groups // STAGE, outer, 0)
        plsc.subcore_barrier()
        pltpu.sync_copy(acc.at[pl.ds(r0, ROWS_PER_TILE)],
                        out_hbm.at[c].at[pl.ds(r0, ROWS_PER_TILE)])

    return agg


def _make_deg(n_groups):
    @functools.partial(
        pl.kernel,
        out_type=jax.ShapeDtypeStruct((2, NPAD, 128), jnp.float32),
        mesh=_sc_mesh(),
        scratch_types=[
            pltpu.VMEM((STAGE, GROUP), jnp.int32),
            pltpu.VMEM((GROUP, 128), jnp.float32),
            pltpu.VMEM((GROUP, 128), jnp.float32),
            pltpu.VMEM_SHARED((NPAD, 128), jnp.float32),
            pltpu.SemaphoreType.DMA,
        ],
    )
    def deg(ones_hbm, zeros_hbm, dst_hbm, out_hbm,
            dst_v, ones_v, zero_v, acc, ssem):
        c = lax.axis_index("c")
        s = lax.axis_index("s")
        r0 = s * ROWS_PER_TILE
        pltpu.sync_copy(zeros_hbm, zero_v)
        _zero_acc_rows(acc, zero_v, r0)
        pltpu.sync_copy(ones_hbm, ones_v)
        plsc.subcore_barrier()

        def outer(o, carry):
            pltpu.sync_copy(dst_hbm.at[c, s, o], dst_v)
            sd = [pltpu.async_copy(ones_v, acc.at[dst_v.at[g]], ssem,
                                   add=True)
                  for g in range(STAGE)]
            for d in sd:
                d.wait()
            return carry

        lax.fori_loop(0, n_groups // STAGE, outer, 0)
        plsc.subcore_barrier()
        pltpu.sync_copy(acc.at[pl.ds(r0, ROWS_PER_TILE)],
                        out_hbm.at[c].at[pl.ds(r0, ROWS_PER_TILE)])

    return deg


def _dis_of(deg_ref):
    return lax.rsqrt(jnp.maximum(deg_ref[:, 0:1], 1.0))


def _tc1_body(deg_ref, x_ref, w_ref, o_ref):
    dis = _dis_of(deg_ref)
    o_ref[...] = jnp.dot(x_ref[...] * dis, w_ref[...],
                         preferred_element_type=jnp.float32)


def _tc2_body(deg_ref, a_ref, b1_ref, w_ref, o_ref):
    dis = _dis_of(deg_ref)
    h = jnp.maximum(a_ref[...] * dis + b1_ref[...], 0.0)
    o_ref[...] = jnp.dot(h, w_ref[...],
                         preferred_element_type=jnp.float32) * dis


def _tc3_body(deg_ref, a_ref, b2_ref, w1_ref, c1_ref, w2_ref, c2_ref, o_ref):
    dis = _dis_of(deg_ref)
    h = jnp.maximum(a_ref[...] * dis + b2_ref[...], 0.0)
    p = jnp.dot(h, w1_ref[...], preferred_element_type=jnp.float32) + c1_ref[...]
    p = jnp.where(p > 0.0, p, jnp.exp(p) - 1.0)
    o_ref[...] = jnp.dot(p, w2_ref[...],
                         preferred_element_type=jnp.float32) + c2_ref[...]


def _row_spec(cols):
    return pl.BlockSpec((_BLK, cols), lambda i: (i, 0))


def _full_spec(r, c):
    return pl.BlockSpec((r, c), lambda i: (0, 0))


def kernel(x, edge_index, W1, b1, W2, b2, Wfc1, bfc1, Wfc2, bfc2):
    i32 = jnp.int32
    f32 = jnp.float32
    loop = jnp.arange(NN, dtype=i32)
    src = jnp.concatenate([edge_index[0], loop])
    dst = jnp.concatenate([edge_index[1], loop])

    pid1 = jnp.arange(EP1 - ET, dtype=i32)
    real1 = jnp.stack([src, src + NN])
    pads1 = jnp.broadcast_to(2 * NN + (pid1 % ZPAD), (2, EP1 - ET))
    src1 = jnp.concatenate([real1, pads1], axis=1).reshape(
        2, 16, G1 // STAGE, STAGE, GROUP)
    dstp1 = jnp.concatenate([dst, pid1 % NPAD])
    dst1 = jnp.broadcast_to(dstp1, (2, EP1)).reshape(
        2, 16, G1 // STAGE, STAGE, GROUP)
    pid2 = jnp.arange(EP2 - ET, dtype=i32)
    srcp2 = jnp.concatenate([src, NN + (pid2 % ZPAD)])
    dstp2 = jnp.concatenate([dst, pid2 % NPAD])
    src2 = srcp2.reshape(2, 16, G2 // STAGE, STAGE, GROUP)
    dst2 = dstp2.reshape(2, 16, G2 // STAGE, STAGE, GROUP)
    dstd = jnp.concatenate([dst, NN + (pid2 % (NPAD - NN))]).reshape(
        2, 16, G2 // STAGE, STAGE, GROUP)

    zeros128 = jnp.zeros((GROUP, 128), f32)
    ones128 = jnp.ones((GROUP, 128), f32)
    ztab = jnp.zeros((ZPAD, 128), f32)

    degp = _make_deg(G2)(ones128, zeros128, dstd)
    degs = (degp[0, :NN, :16] + degp[1, :NN, :16])

    xs1 = pl.pallas_call(
        _tc1_body,
        grid=(NN // _BLK,),
        in_specs=[_row_spec(16), _row_spec(128), _full_spec(128, 256)],
        out_specs=_row_spec(256),
        out_shape=jax.ShapeDtypeStruct((NN, 256), f32),
    )(degs, x, W1)

    table1 = jnp.concatenate([xs1[:, :128], xs1[:, 128:], ztab], axis=0)
    agg1 = _make_agg(2 * NN + ZPAD, G1)(table1, zeros128, src1, dst1)
    agg1c = jnp.concatenate([agg1[0, :NN], agg1[1, :NN]], axis=1)

    xs2 = pl.pallas_call(
        _tc2_body,
        grid=(NN // _BLK,),
        in_specs=[_row_spec(16), _row_spec(256), _full_spec(1, 256),
                  _full_spec(256, 128)],
        out_specs=_row_spec(128),
        out_shape=jax.ShapeDtypeStruct((NN, 128), f32),
    )(degs, agg1c, b1.reshape(1, 256), W2)

    table2 = jnp.concatenate([xs2, ztab], axis=0)
    agg2 = _make_agg(NN + ZPAD, G2)(table2, zeros128, src2, dst2)
    agg2s = agg2[0, :NN] + agg2[1, :NN]

    out = pl.pallas_call(
        _tc3_body,
        grid=(NN // _BLK,),
        in_specs=[_row_spec(16), _row_spec(128), _full_spec(1, 128),
                  _full_spec(128, 128), _full_spec(1, 128),
                  _full_spec(128, 128), _full_spec(1, 128)],
        out_specs=_row_spec(128),
        out_shape=jax.ShapeDtypeStruct((NN, 128), f32),
    )(degs, agg2s, b2.reshape(1, 128), Wfc1, bfc1.reshape(1, 128),
      Wfc2, bfc2.reshape(1, 128))
    return out


# speedup vs baseline: 17.8655x; 1.0541x over previous
import functools

import jax
import jax.numpy as jnp
from jax import lax
from jax.experimental import pallas as pl
from jax.experimental.pallas import tpu as pltpu
from jax.experimental.pallas import tpu_sc as plsc

NN = 10000
NE = 320000
ET = NE + NN
NPAD = 10112
DUMMY = NN
GROUP = 128
STAGE = 8
G1 = 168
G2 = 88
EP1 = 16 * G1 * GROUP
EP2 = 32 * G2 * GROUP
ROWS_PER_TILE = NPAD // 16
ZPAD = 1000
_BLK = 1000


def _sc_mesh():
    return plsc.VectorSubcoreMesh(core_axis_name="c", subcore_axis_name="s",
                                  num_cores=2, num_subcores=16)


def _zero_acc_rows(acc, zbuf, r0):
    for t in range(ROWS_PER_TILE // GROUP):
        pltpu.sync_copy(zbuf, acc.at[pl.ds(r0 + t * GROUP, GROUP)])
    rem = ROWS_PER_TILE % GROUP
    if rem:
        base = r0 + (ROWS_PER_TILE // GROUP) * GROUP
        pltpu.sync_copy(zbuf.at[pl.ds(0, rem)], acc.at[pl.ds(base, rem)])


def _make_agg(n_table_rows, n_groups):
    @functools.partial(
        pl.kernel,
        out_type=jax.ShapeDtypeStruct((2, NPAD, 128), jnp.float32),
        mesh=_sc_mesh(),
        scratch_types=[
            pltpu.VMEM((STAGE, GROUP), jnp.int32),
            pltpu.VMEM((STAGE, GROUP), jnp.int32),
            pltpu.VMEM((GROUP, 128), jnp.float32),
            pltpu.VMEM((GROUP, 128), jnp.float32),
            pltpu.VMEM_SHARED((NPAD, 128), jnp.float32),
            pltpu.SemaphoreType.DMA,
            pltpu.SemaphoreType.DMA,
            pltpu.SemaphoreType.DMA,
            pltpu.SemaphoreType.DMA,
        ],
    )
    def agg(table_hbm, zeros_hbm, src_hbm, dst_hbm, out_hbm,
            src_v, dst_v, val0, val1, acc, gsem0, gsem1, ssem0, ssem1):
        c = lax.axis_index("c")
        s = lax.axis_index("s")
        r0 = s * ROWS_PER_TILE
        pltpu.sync_copy(zeros_hbm, val0)
        _zero_acc_rows(acc, val0, r0)
        plsc.subcore_barrier()
        vals = (val0, val1)
        gsems = (gsem0, gsem1)
        ssems = (ssem0, ssem1)

        def outer(o, carry):
            pltpu.sync_copy(src_hbm.at[c, s, o], src_v)
            pltpu.sync_copy(dst_hbm.at[c, s, o], dst_v)
            sd = [None] * STAGE
            for g in range(STAGE):
                b = g & 1
                if g >= 2:
                    sd[g - 2].wait()
                gd = pltpu.async_copy(table_hbm.at[src_v.at[g]], vals[b],
                                      gsems[b])
                gd.wait()
                sd[g] = pltpu.async_copy(vals[b], acc.at[dst_v.at[g]],
                                         ssems[b], add=True)
            sd[STAGE - 2].wait()
            sd[STAGE - 1].wait()
            return carry

        lax.fori_loop(0, n_groups // STAGE, outer, 0)
        plsc.subcore_barrier()
        pltpu.sync_copy(acc.at[pl.ds(r0, ROWS_PER_TILE)],
                        out_hbm.at[c].at[pl.ds(r0, ROWS_PER_TILE)])

    return agg


def _make_deg(n_groups):
    @functools.partial(
        pl.kernel,
        out_type=jax.ShapeDtypeStruct((2, NPAD, 128), jnp.float32),
        mesh=_sc_mesh(),
        scratch_types=[
            pltpu.VMEM((STAGE, GROUP), jnp.int32),
            pltpu.VMEM((GROUP, 128), jnp.float32),
            pltpu.VMEM((GROUP, 128), jnp.float32),
            pltpu.VMEM_SHARED((NPAD, 128), jnp.float32),
            pltpu.SemaphoreType.DMA,
        ],
    )
    def deg(ones_hbm, zeros_hbm, dst_hbm, out_hbm,
            dst_v, ones_v, zero_v, acc, ssem):
        c = lax.axis_index("c")
        s = lax.axis_index("s")
        r0 = s * ROWS_PER_TILE
        pltpu.sync_copy(zeros_hbm, zero_v)
        _zero_acc_rows(acc, zero_v, r0)
        pltpu.sync_copy(ones_hbm, ones_v)
        plsc.subcore_barrier()

        def outer(o, carry):
            pltpu.sync_copy(dst_hbm.at[c, s, o], dst_v)
            sd = [pltpu.async_copy(ones_v, acc.at[dst_v.at[g]], ssem,
                                   add=True)
                  for g in range(STAGE)]
            for d in sd:
                d.wait()
            return carry

        lax.fori_loop(0, n_groups // STAGE, outer, 0)
        plsc.subcore_barrier()
        pltpu.sync_copy(acc.at[pl.ds(r0, ROWS_PER_TILE)],
                        out_hbm.at[c].at[pl.ds(r0, ROWS_PER_TILE)])

    return deg


def _dis_of(deg_ref):
    return lax.rsqrt(jnp.maximum(deg_ref[:, 0:1], 1.0))


def _tc1_body(deg_ref, x_ref, w_ref, o_ref):
    i = pl.program_id(0)

    @pl.when(i < 20)
    def _():
        dis = _dis_of(deg_ref)
        o_ref[...] = jnp.dot(x_ref[...] * dis, w_ref[...],
                             preferred_element_type=jnp.float32)

    @pl.when(i >= 20)
    def _():
        o_ref[...] = jnp.zeros_like(o_ref)


def _tc2_body(deg_ref, a0_ref, a1_ref, b1_ref, w_ref, o_ref):
    i = pl.program_id(0)

    @pl.when(i < 10)
    def _():
        dis = _dis_of(deg_ref)
        h = jnp.concatenate([a0_ref[0], a1_ref[0]], axis=1) * dis + b1_ref[...]
        h = jnp.maximum(h, 0.0)
        o_ref[...] = jnp.dot(h, w_ref[...],
                             preferred_element_type=jnp.float32) * dis

    @pl.when(i >= 10)
    def _():
        o_ref[...] = jnp.zeros_like(o_ref)


def _tc3_body(deg_ref, a0_ref, a1_ref, b2_ref, w1_ref, c1_ref, w2_ref,
              c2_ref, o_ref):
    dis = _dis_of(deg_ref)
    h = jnp.maximum((a0_ref[0] + a1_ref[0]) * dis + b2_ref[...], 0.0)
    p = jnp.dot(h, w1_ref[...], preferred_element_type=jnp.float32) + c1_ref[...]
    p = jnp.where(p > 0.0, p, jnp.exp(p) - 1.0)
    o_ref[...] = jnp.dot(p, w2_ref[...],
                         preferred_element_type=jnp.float32) + c2_ref[...]


def _row_spec(cols, mod=None):
    if mod is None:
        return pl.BlockSpec((_BLK, cols), lambda i: (i, 0))
    return pl.BlockSpec((_BLK, cols), lambda i: (i % mod, 0))


def _core_spec(cols, core):
    return pl.BlockSpec((1, _BLK, cols), lambda i: (core, i, 0))


def _full_spec(r, c):
    return pl.BlockSpec((r, c), lambda i: (0, 0))


def kernel(x, edge_index, W1, b1, W2, b2, Wfc1, bfc1, Wfc2, bfc2):
    i32 = jnp.int32
    f32 = jnp.float32
    loop = jnp.arange(NN, dtype=i32)
    src = jnp.concatenate([edge_index[0], loop])
    dst = jnp.concatenate([edge_index[1], loop])

    pid1 = jnp.arange(EP1 - ET, dtype=i32)
    real1 = jnp.stack([src, src + NN])
    pads1 = jnp.broadcast_to(2 * NN + (pid1 % ZPAD), (2, EP1 - ET))
    src1 = jnp.concatenate([real1, pads1], axis=1).reshape(
        2, 16, G1 // STAGE, STAGE, GROUP)
    dstp1 = jnp.concatenate([dst, pid1 % NPAD])
    dst1 = jnp.broadcast_to(dstp1, (2, EP1)).reshape(
        2, 16, G1 // STAGE, STAGE, GROUP)
    pid2 = jnp.arange(EP2 - ET, dtype=i32)
    srcp2 = jnp.concatenate([src, NN + (pid2 % ZPAD)])
    dstp2 = jnp.concatenate([dst, pid2 % NPAD])
    src2 = srcp2.reshape(2, 16, G2 // STAGE, STAGE, GROUP)
    dst2 = dstp2.reshape(2, 16, G2 // STAGE, STAGE, GROUP)
    dstd = jnp.concatenate([dst, NN + (pid2 % (NPAD - NN))]).reshape(
        2, 16, G2 // STAGE, STAGE, GROUP)

    zeros128 = jnp.zeros((GROUP, 128), f32)
    ones128 = jnp.ones((GROUP, 128), f32)

    degp = _make_deg(G2)(ones128, zeros128, dstd)
    degs = degp[0, :NN, :16] + degp[1, :NN, :16]

    table1 = pl.pallas_call(
        _tc1_body,
        grid=(21,),
        in_specs=[_row_spec(16, mod=10), _row_spec(128, mod=10),
                  pl.BlockSpec((128, 128), lambda i: (0, (i // 10) % 2))],
        out_specs=pl.BlockSpec((_BLK, 128), lambda i: (i, 0)),
        out_shape=jax.ShapeDtypeStruct((2 * NN + ZPAD, 128), f32),
    )(degs, x, W1)

    agg1 = _make_agg(2 * NN + ZPAD, G1)(table1, zeros128, src1, dst1)

    table2 = pl.pallas_call(
        _tc2_body,
        grid=(11,),
        in_specs=[_row_spec(16, mod=10),
                  pl.BlockSpec((1, _BLK, 128), lambda i: (0, i % 10, 0)),
                  pl.BlockSpec((1, _BLK, 128), lambda i: (1, i % 10, 0)),
                  _full_spec(1, 256), _full_spec(256, 128)],
        out_specs=pl.BlockSpec((_BLK, 128), lambda i: (i, 0)),
        out_shape=jax.ShapeDtypeStruct((NN + ZPAD, 128), f32),
    )(degs, agg1, agg1, b1.reshape(1, 256), W2)

    agg2 = _make_agg(NN + ZPAD, G2)(table2, zeros128, src2, dst2)

    out = pl.pallas_call(
        _tc3_body,
        grid=(NN // _BLK,),
        in_specs=[_row_spec(16),
                  pl.BlockSpec((1, _BLK, 128), lambda i: (0, i, 0)),
                  pl.BlockSpec((1, _BLK, 128), lambda i: (1, i, 0)),
                  _full_spec(1, 128),
                  _full_spec(128, 128), _full_spec(1, 128),
                  _full_spec(128, 128), _full_spec(1, 128)],
        out_specs=_row_spec(128),
        out_shape=jax.ShapeDtypeStruct((NN, 128), f32),
    )(degs, agg2, agg2, b2.reshape(1, 128), Wfc1, bfc1.reshape(1, 128),
      Wfc2, bfc2.reshape(1, 128))
    return out


# speedup vs baseline: 18.0941x; 1.0128x over previous
import functools

import jax
import jax.numpy as jnp
from jax import lax
from jax.experimental import pallas as pl
from jax.experimental.pallas import tpu as pltpu
from jax.experimental.pallas import tpu_sc as plsc

NN = 10000
NE = 320000
ET = NE + NN
NPAD = 10112
DUMMY = NN
GROUP = 128
STAGE = 8
G1 = 168
G2 = 88
EP1 = 16 * G1 * GROUP
EP2 = 32 * G2 * GROUP
ROWS_PER_TILE = NPAD // 16
ZPAD = 2000
_BLK2 = 2000
_BLK = 1000


def _sc_mesh():
    return plsc.VectorSubcoreMesh(core_axis_name="c", subcore_axis_name="s",
                                  num_cores=2, num_subcores=16)


def _zero_acc_rows(acc, zbuf, r0):
    for t in range(ROWS_PER_TILE // GROUP):
        pltpu.sync_copy(zbuf, acc.at[pl.ds(r0 + t * GROUP, GROUP)])
    rem = ROWS_PER_TILE % GROUP
    if rem:
        base = r0 + (ROWS_PER_TILE // GROUP) * GROUP
        pltpu.sync_copy(zbuf.at[pl.ds(0, rem)], acc.at[pl.ds(base, rem)])


def _make_agg(n_table_rows, n_groups):
    @functools.partial(
        pl.kernel,
        out_type=jax.ShapeDtypeStruct((2, NPAD, 128), jnp.float32),
        mesh=_sc_mesh(),
        scratch_types=[
            pltpu.VMEM((STAGE, GROUP), jnp.int32),
            pltpu.VMEM((STAGE, GROUP), jnp.int32),
            pltpu.VMEM((GROUP, 128), jnp.float32),
            pltpu.VMEM((GROUP, 128), jnp.float32),
            pltpu.VMEM_SHARED((NPAD, 128), jnp.float32),
            pltpu.SemaphoreType.DMA,
            pltpu.SemaphoreType.DMA,
            pltpu.SemaphoreType.DMA,
            pltpu.SemaphoreType.DMA,
        ],
    )
    def agg(table_hbm, zeros_hbm, src_hbm, dst_hbm, out_hbm,
            src_v, dst_v, val0, val1, acc, gsem0, gsem1, ssem0, ssem1):
        c = lax.axis_index("c")
        s = lax.axis_index("s")
        r0 = s * ROWS_PER_TILE
        pltpu.sync_copy(zeros_hbm, val0)
        _zero_acc_rows(acc, val0, r0)
        plsc.subcore_barrier()
        vals = (val0, val1)
        gsems = (gsem0, gsem1)
        ssems = (ssem0, ssem1)

        def outer(o, carry):
            pltpu.sync_copy(src_hbm.at[c, s, o], src_v)
            pltpu.sync_copy(dst_hbm.at[c, s, o], dst_v)
            sd = [None] * STAGE
            for g in range(STAGE):
                b = g & 1
                if g >= 2:
                    sd[g - 2].wait()
                gd = pltpu.async_copy(table_hbm.at[src_v.at[g]], vals[b],
                                      gsems[b])
                gd.wait()
                sd[g] = pltpu.async_copy(vals[b], acc.at[dst_v.at[g]],
                                         ssems[b], add=True)
            sd[STAGE - 2].wait()
            sd[STAGE - 1].wait()
            return carry

        lax.fori_loop(0, n_groups // STAGE, outer, 0)
        plsc.subcore_barrier()
        pltpu.sync_copy(acc.at[pl.ds(r0, ROWS_PER_TILE)],
                        out_hbm.at[c].at[pl.ds(r0, ROWS_PER_TILE)])

    return agg


def _make_deg(n_groups):
    @functools.partial(
        pl.kernel,
        out_type=jax.ShapeDtypeStruct((2, NPAD, 128), jnp.float32),
        mesh=_sc_mesh(),
        scratch_types=[
            pltpu.VMEM((STAGE, GROUP), jnp.int32),
            pltpu.VMEM((GROUP, 128), jnp.float32),
            pltpu.VMEM((GROUP, 128), jnp.float32),
            pltpu.VMEM_SHARED((NPAD, 128), jnp.float32),
            pltpu.SemaphoreType.DMA,
        ],
    )
    def deg(ones_hbm, zeros_hbm, dst_hbm, out_hbm,
            dst_v, ones_v, zero_v, acc, ssem):
        c = lax.axis_index("c")
        s = lax.axis_index("s")
        r0 = s * ROWS_PER_TILE
        pltpu.sync_copy(zeros_hbm, zero_v)
        _zero_acc_rows(acc, zero_v, r0)
        pltpu.sync_copy(ones_hbm, ones_v)
        plsc.subcore_barrier()

        def outer(o, carry):
            pltpu.sync_copy(dst_hbm.at[c, s, o], dst_v)
            sd = [pltpu.async_copy(ones_v, acc.at[dst_v.at[g]], ssem,
                                   add=True)
                  for g in range(STAGE)]
            for d in sd:
                d.wait()
            return carry

        lax.fori_loop(0, n_groups // STAGE, outer, 0)
        plsc.subcore_barrier()
        pltpu.sync_copy(acc.at[pl.ds(r0, ROWS_PER_TILE)],
                        out_hbm.at[c].at[pl.ds(r0, ROWS_PER_TILE)])

    return deg


def _dis_of(deg_ref):
    return lax.rsqrt(jnp.maximum(deg_ref[:, 0:1], 1.0))


def _tc1_body(deg_ref, x_ref, w_ref, o_ref):
    i = pl.program_id(0)

    @pl.when(i < 10)
    def _():
        dis = _dis_of(deg_ref)
        o_ref[...] = jnp.dot(x_ref[...] * dis, w_ref[...],
                             preferred_element_type=jnp.float32)

    @pl.when(i >= 10)
    def _():
        o_ref[...] = jnp.zeros_like(o_ref)


def _tc2_body(deg_ref, a0_ref, a1_ref, b1_ref, w_ref, o_ref):
    i = pl.program_id(0)

    @pl.when(i < 5)
    def _():
        dis = _dis_of(deg_ref)
        h = jnp.concatenate([a0_ref[0], a1_ref[0]], axis=1) * dis + b1_ref[...]
        h = jnp.maximum(h, 0.0)
        o_ref[...] = jnp.dot(h, w_ref[...],
                             preferred_element_type=jnp.float32) * dis

    @pl.when(i >= 5)
    def _():
        o_ref[...] = jnp.zeros_like(o_ref)


def _tc3_body(deg_ref, a0_ref, a1_ref, b2_ref, w1_ref, c1_ref, w2_ref,
              c2_ref, o_ref):
    dis = _dis_of(deg_ref)
    h = jnp.maximum((a0_ref[0] + a1_ref[0]) * dis + b2_ref[...], 0.0)
    p = jnp.dot(h, w1_ref[...], preferred_element_type=jnp.float32) + c1_ref[...]
    p = jnp.where(p > 0.0, p, jnp.exp(p) - 1.0)
    o_ref[...] = jnp.dot(p, w2_ref[...],
                         preferred_element_type=jnp.float32) + c2_ref[...]


def _row_spec(cols, mod=None):
    if mod is None:
        return pl.BlockSpec((_BLK, cols), lambda i: (i, 0))
    return pl.BlockSpec((_BLK, cols), lambda i: (i % mod, 0))


def _core_spec(cols, core):
    return pl.BlockSpec((1, _BLK, cols), lambda i: (core, i, 0))


def _full_spec(r, c):
    return pl.BlockSpec((r, c), lambda i: (0, 0))


def kernel(x, edge_index, W1, b1, W2, b2, Wfc1, bfc1, Wfc2, bfc2):
    i32 = jnp.int32
    f32 = jnp.float32
    loop = jnp.arange(NN, dtype=i32)
    src = jnp.concatenate([edge_index[0], loop])
    dst = jnp.concatenate([edge_index[1], loop])

    pid1 = jnp.arange(EP1 - ET, dtype=i32)
    real1 = jnp.stack([src, src + NN])
    pads1 = jnp.broadcast_to(2 * NN + (pid1 % ZPAD), (2, EP1 - ET))
    src1 = jnp.concatenate([real1, pads1], axis=1).reshape(
        2, 16, G1 // STAGE, STAGE, GROUP)
    dstp1 = jnp.concatenate([dst, pid1 % NPAD])
    dst1 = jnp.broadcast_to(dstp1, (2, EP1)).reshape(
        2, 16, G1 // STAGE, STAGE, GROUP)
    pid2 = jnp.arange(EP2 - ET, dtype=i32)
    srcp2 = jnp.concatenate([src, NN + (pid2 % ZPAD)])
    dstp2 = jnp.concatenate([dst, pid2 % NPAD])
    src2 = srcp2.reshape(2, 16, G2 // STAGE, STAGE, GROUP)
    dst2 = dstp2.reshape(2, 16, G2 // STAGE, STAGE, GROUP)
    dstd = jnp.concatenate([dst, NN + (pid2 % (NPAD - NN))]).reshape(
        2, 16, G2 // STAGE, STAGE, GROUP)

    zeros128 = jnp.zeros((GROUP, 128), f32)
    ones128 = jnp.ones((GROUP, 128), f32)

    degp = _make_deg(G2)(ones128, zeros128, dstd)
    degs = degp[0, :NN, :16] + degp[1, :NN, :16]

    table1 = pl.pallas_call(
        _tc1_body,
        grid=(11,),
        in_specs=[pl.BlockSpec((_BLK2, 16), lambda i: (i % 5, 0)),
                  pl.BlockSpec((_BLK2, 128), lambda i: (i % 5, 0)),
                  pl.BlockSpec((128, 128), lambda i: (0, (i // 5) % 2))],
        out_specs=pl.BlockSpec((_BLK2, 128), lambda i: (i, 0)),
        out_shape=jax.ShapeDtypeStruct((2 * NN + ZPAD, 128), f32),
    )(degs, x, W1)

    agg1 = _make_agg(2 * NN + ZPAD, G1)(table1, zeros128, src1, dst1)

    table2 = pl.pallas_call(
        _tc2_body,
        grid=(6,),
        in_specs=[pl.BlockSpec((_BLK2, 16), lambda i: (i % 5, 0)),
                  pl.BlockSpec((1, _BLK2, 128), lambda i: (0, i % 5, 0)),
                  pl.BlockSpec((1, _BLK2, 128), lambda i: (1, i % 5, 0)),
                  _full_spec(1, 256), _full_spec(256, 128)],
        out_specs=pl.BlockSpec((_BLK2, 128), lambda i: (i, 0)),
        out_shape=jax.ShapeDtypeStruct((NN + ZPAD, 128), f32),
    )(degs, agg1, agg1, b1.reshape(1, 256), W2)

    agg2 = _make_agg(NN + ZPAD, G2)(table2, zeros128, src2, dst2)

    out = pl.pallas_call(
        _tc3_body,
        grid=(NN // _BLK,),
        in_specs=[_row_spec(16),
                  pl.BlockSpec((1, _BLK, 128), lambda i: (0, i, 0)),
                  pl.BlockSpec((1, _BLK, 128), lambda i: (1, i, 0)),
                  _full_spec(1, 128),
                  _full_spec(128, 128), _full_spec(1, 128),
                  _full_spec(128, 128), _full_spec(1, 128)],
        out_specs=_row_spec(128),
        out_shape=jax.ShapeDtypeStruct((NN, 128), f32),
    )(degs, agg2, agg2, b2.reshape(1, 128), Wfc1, bfc1.reshape(1, 128),
      Wfc2, bfc2.reshape(1, 128))
    return out


# speedup vs baseline: 18.1398x; 1.0025x over previous
import functools

import jax
import jax.numpy as jnp
from jax import lax
from jax.experimental import pallas as pl
from jax.experimental.pallas import tpu as pltpu
from jax.experimental.pallas import tpu_sc as plsc

NN = 10000
NE = 320000
ET = NE + NN
NPAD = 10112
DUMMY = NN
GROUP = 128
STAGE = 8
G1 = 168
G2 = 88
EP1 = 16 * G1 * GROUP
EP2 = 32 * G2 * GROUP
ROWS_PER_TILE = NPAD // 16
ZPAD = 2000
_BLK2 = 2000
_BLK = 1000


def _sc_mesh():
    return plsc.VectorSubcoreMesh(core_axis_name="c", subcore_axis_name="s",
                                  num_cores=2, num_subcores=16)


def _zero_acc_rows(acc, zbuf, r0):
    for t in range(ROWS_PER_TILE // GROUP):
        pltpu.sync_copy(zbuf, acc.at[pl.ds(r0 + t * GROUP, GROUP)])
    rem = ROWS_PER_TILE % GROUP
    if rem:
        base = r0 + (ROWS_PER_TILE // GROUP) * GROUP
        pltpu.sync_copy(zbuf.at[pl.ds(0, rem)], acc.at[pl.ds(base, rem)])


def _make_agg(n_table_rows, n_groups):
    @functools.partial(
        pl.kernel,
        out_type=jax.ShapeDtypeStruct((2, NPAD, 128), jnp.float32),
        mesh=_sc_mesh(),
        scratch_types=[
            pltpu.VMEM((STAGE, GROUP), jnp.int32),
            pltpu.VMEM((STAGE, GROUP), jnp.int32),
            pltpu.VMEM((GROUP, 128), jnp.float32),
            pltpu.VMEM((GROUP, 128), jnp.float32),
            pltpu.VMEM_SHARED((NPAD, 128), jnp.float32),
            pltpu.SemaphoreType.DMA,
            pltpu.SemaphoreType.DMA,
            pltpu.SemaphoreType.DMA,
            pltpu.SemaphoreType.DMA,
        ],
    )
    def agg(table_hbm, zeros_hbm, src_hbm, dst_hbm, out_hbm,
            src_v, dst_v, val0, val1, acc, gsem0, gsem1, ssem0, ssem1):
        c = lax.axis_index("c")
        s = lax.axis_index("s")
        r0 = s * ROWS_PER_TILE
        pltpu.sync_copy(zeros_hbm, val0)
        _zero_acc_rows(acc, val0, r0)
        plsc.subcore_barrier()
        vals = (val0, val1)
        gsems = (gsem0, gsem1)
        ssems = (ssem0, ssem1)

        def outer(o, carry):
            pltpu.sync_copy(src_hbm.at[c, s, o], src_v)
            pltpu.sync_copy(dst_hbm.at[c, s, o], dst_v)
            sd = [None] * STAGE
            for g in range(STAGE):
                b = g & 1
                if g >= 2:
                    sd[g - 2].wait()
                gd = pltpu.async_copy(table_hbm.at[src_v.at[g]], vals[b],
                                      gsems[b])
                gd.wait()
                sd[g] = pltpu.async_copy(vals[b], acc.at[dst_v.at[g]],
                                         ssems[b], add=True)
            sd[STAGE - 2].wait()
            sd[STAGE - 1].wait()
            return carry

        lax.fori_loop(0, n_groups // STAGE, outer, 0)
        plsc.subcore_barrier()
        pltpu.sync_copy(acc.at[pl.ds(r0, ROWS_PER_TILE)],
                        out_hbm.at[c].at[pl.ds(r0, ROWS_PER_TILE)])

    return agg


def _make_deg(n_groups):
    @functools.partial(
        pl.kernel,
        out_type=jax.ShapeDtypeStruct((2, NPAD, 128), jnp.float32),
        mesh=_sc_mesh(),
        scratch_types=[
            pltpu.VMEM((STAGE, GROUP), jnp.int32),
            pltpu.VMEM((GROUP, 128), jnp.float32),
            pltpu.VMEM((GROUP, 128), jnp.float32),
            pltpu.VMEM_SHARED((NPAD, 128), jnp.float32),
            pltpu.SemaphoreType.DMA,
        ],
    )
    def deg(ones_hbm, zeros_hbm, dst_hbm, out_hbm,
            dst_v, ones_v, zero_v, acc, ssem):
        c = lax.axis_index("c")
        s = lax.axis_index("s")
        r0 = s * ROWS_PER_TILE
        pltpu.sync_copy(zeros_hbm, zero_v)
        _zero_acc_rows(acc, zero_v, r0)
        pltpu.sync_copy(ones_hbm, ones_v)
        plsc.subcore_barrier()

        def outer(o, carry):
            pltpu.sync_copy(dst_hbm.at[c, s, o], dst_v)
            sd = [pltpu.async_copy(ones_v, acc.at[dst_v.at[g]], ssem,
                                   add=True)
                  for g in range(STAGE)]
            for d in sd:
                d.wait()
            return carry

        lax.fori_loop(0, n_groups // STAGE, outer, 0)
        plsc.subcore_barrier()
        pltpu.sync_copy(acc.at[pl.ds(r0, ROWS_PER_TILE)],
                        out_hbm.at[c].at[pl.ds(r0, ROWS_PER_TILE)])

    return deg


def _dis_of(deg_ref):
    return lax.rsqrt(jnp.maximum(deg_ref[:, 0:1], 1.0))


def _tc1_body(deg_ref, x_ref, w_ref, o_ref):
    i = pl.program_id(0)

    @pl.when(i < 10)
    def _():
        dis = _dis_of(deg_ref)
        o_ref[...] = jnp.dot(x_ref[...] * dis, w_ref[...],
                             preferred_element_type=jnp.float32)

    @pl.when(i >= 10)
    def _():
        o_ref[...] = jnp.zeros_like(o_ref)


def _tc2_body(deg_ref, a0_ref, a1_ref, b1_ref, w_ref, o_ref):
    i = pl.program_id(0)

    @pl.when(i < 5)
    def _():
        dis = _dis_of(deg_ref)
        h = jnp.concatenate([a0_ref[0], a1_ref[0]], axis=1) * dis + b1_ref[...]
        h = jnp.maximum(h, 0.0)
        o_ref[...] = jnp.dot(h, w_ref[...],
                             preferred_element_type=jnp.float32) * dis

    @pl.when(i >= 5)
    def _():
        o_ref[...] = jnp.zeros_like(o_ref)


def _tc3_body(deg_ref, a0_ref, a1_ref, b2_ref, w1_ref, c1_ref, w2_ref,
              c2_ref, o_ref):
    dis = _dis_of(deg_ref)
    h = jnp.maximum((a0_ref[0] + a1_ref[0]) * dis + b2_ref[...], 0.0)
    p = jnp.dot(h, w1_ref[...], preferred_element_type=jnp.float32) + c1_ref[...]
    p = jnp.where(p > 0.0, p, jnp.exp(p) - 1.0)
    o_ref[...] = jnp.dot(p, w2_ref[...],
                         preferred_element_type=jnp.float32) + c2_ref[...]


def _row_spec(cols):
    return pl.BlockSpec((_BLK, cols), lambda i: (i, 0))


def _full_spec(r, c):
    return pl.BlockSpec((r, c), lambda i: (0, 0))


def kernel(x, edge_index, W1, b1, W2, b2, Wfc1, bfc1, Wfc2, bfc2):
    i32 = jnp.int32
    f32 = jnp.float32
    loop = jnp.arange(NN, dtype=i32)
    src = jnp.concatenate([edge_index[0], loop])
    dst = jnp.concatenate([edge_index[1], loop])

    pid1 = jnp.arange(EP1 - ET, dtype=i32)
    real1 = jnp.stack([src, src + NN])
    pads1 = jnp.broadcast_to(2 * NN + (pid1 % ZPAD), (2, EP1 - ET))
    src1 = jnp.concatenate([real1, pads1], axis=1).reshape(
        2, 16, G1 // STAGE, STAGE, GROUP)
    dstp1 = jnp.concatenate([dst, pid1 % NPAD])
    dst1 = jnp.broadcast_to(dstp1, (2, EP1)).reshape(
        2, 16, G1 // STAGE, STAGE, GROUP)
    pid2 = jnp.arange(EP2 - ET, dtype=i32)
    srcp2 = jnp.concatenate([src, NN + (pid2 % ZPAD)])
    dstp2 = jnp.concatenate([dst, pid2 % NPAD])
    src2 = srcp2.reshape(2, 16, G2 // STAGE, STAGE, GROUP)
    dst2 = dstp2.reshape(2, 16, G2 // STAGE, STAGE, GROUP)
    dstd = jnp.concatenate([dst, NN + (pid2 % (NPAD - NN))]).reshape(
        2, 16, G2 // STAGE, STAGE, GROUP)

    zeros128 = jnp.zeros((GROUP, 128), f32)
    ones128 = jnp.ones((GROUP, 128), f32)

    degp = _make_deg(G2)(ones128, zeros128, dstd)
    degs = degp[0, :NN, :16] + degp[1, :NN, :16]

    table1 = pl.pallas_call(
        _tc1_body,
        grid=(11,),
        in_specs=[pl.BlockSpec((_BLK2, 16), lambda i: (i % 5, 0)),
                  pl.BlockSpec((_BLK2, 128), lambda i: (i % 5, 0)),
                  pl.BlockSpec((128, 128), lambda i: (0, (i // 5) % 2))],
        out_specs=pl.BlockSpec((_BLK2, 128), lambda i: (i, 0)),
        out_shape=jax.ShapeDtypeStruct((2 * NN + ZPAD, 128), f32),
    )(degs, x, W1)

    agg1 = _make_agg(2 * NN + ZPAD, G1)(table1, zeros128, src1, dst1)

    table2 = pl.pallas_call(
        _tc2_body,
        grid=(6,),
        in_specs=[pl.BlockSpec((_BLK2, 16), lambda i: (i % 5, 0)),
                  pl.BlockSpec((1, _BLK2, 128), lambda i: (0, i % 5, 0)),
                  pl.BlockSpec((1, _BLK2, 128), lambda i: (1, i % 5, 0)),
                  _full_spec(1, 256), _full_spec(256, 128)],
        out_specs=pl.BlockSpec((_BLK2, 128), lambda i: (i, 0)),
        out_shape=jax.ShapeDtypeStruct((NN + ZPAD, 128), f32),
    )(degs, agg1, agg1, b1.reshape(1, 256), W2)

    agg2 = _make_agg(NN + ZPAD, G2)(table2, zeros128, src2, dst2)

    out = pl.pallas_call(
        _tc3_body,
        grid=(NN // _BLK,),
        in_specs=[_row_spec(16),
                  pl.BlockSpec((1, _BLK, 128), lambda i: (0, i, 0)),
                  pl.BlockSpec((1, _BLK, 128), lambda i: (1, i, 0)),
                  _full_spec(1, 128),
                  _full_spec(128, 128), _full_spec(1, 128),
                  _full_spec(128, 128), _full_spec(1, 128)],
        out_specs=_row_spec(128),
        out_shape=jax.ShapeDtypeStruct((NN, 128), f32),
    )(degs, agg2, agg2, b2.reshape(1, 128), Wfc1, bfc1.reshape(1, 128),
      Wfc2, bfc2.reshape(1, 128))
    return out
